# Initial kernel scaffold; baseline (speedup 1.0000x reference)
import jax
import jax.numpy as jnp
from jax.experimental import pallas as pl


def kernel(rbf, sph, idx_sph, weight):
    raise NotImplementedError("write your pallas kernel here")


# speedup vs baseline: 27.5876x; 27.5876x over previous
import functools

import jax
import jax.numpy as jnp
from jax import lax
from jax.experimental import pallas as pl
from jax.experimental.pallas import tpu as pltpu
from jax.experimental.pallas import tpu_sc as plsc

NUM_RADIAL = 128
NUM_SPH = 8
EMB = 32
OUT_COLS = NUM_SPH * EMB

_NC = 2
_NS = 16
_NW = _NC * _NS
_CHUNK = 128


def _sc_gather(table, idx, t_pad, nchunks):
    mesh = plsc.VectorSubcoreMesh(core_axis_name="c", subcore_axis_name="s")

    @functools.partial(
        pl.kernel,
        mesh=mesh,
        out_type=jax.ShapeDtypeStruct((t_pad, NUM_RADIAL), jnp.float32),
        scratch_types=[
            pltpu.VMEM((_CHUNK,), jnp.int32),
            pltpu.VMEM((_CHUNK, NUM_RADIAL), jnp.float32),
            pltpu.SemaphoreType.DMA,
        ],
    )
    def k(table_hbm, idx_hbm, out_hbm, idx_v, rows_v, sem):
        wid = lax.axis_index("s") * _NC + lax.axis_index("c")
        base = wid * nchunks * _CHUNK

        def body(i, carry):
            off = base + i * _CHUNK
            pltpu.sync_copy(idx_hbm.at[pl.ds(off, _CHUNK)], idx_v)
            pltpu.async_copy(table_hbm.at[idx_v], rows_v, sem).wait()
            pltpu.sync_copy(rows_v, out_hbm.at[pl.ds(off, _CHUNK)])
            return carry

        lax.fori_loop(0, nchunks, body, 0, unroll=False)

    return k(table, idx)


def _tc_contract(g, sph, w, b_mat, p_mat, t_pad, tile):
    def body(g_ref, s_ref, w_ref, b_ref, p_ref, o_ref):
        h = jnp.dot(g_ref[...], w_ref[...], preferred_element_type=jnp.float32)
        srep = jnp.dot(s_ref[...], b_ref[...], preferred_element_type=jnp.float32)
        o_ref[...] = jnp.dot(h * srep, p_ref[...],
                             preferred_element_type=jnp.float32)

    return pl.pallas_call(
        body,
        grid=(t_pad // tile,),
        in_specs=[
            pl.BlockSpec((tile, NUM_RADIAL), lambda i: (i, 0)),
            pl.BlockSpec((tile, NUM_SPH), lambda i: (i, 0)),
            pl.BlockSpec((NUM_RADIAL, OUT_COLS), lambda i: (0, 0)),
            pl.BlockSpec((NUM_SPH, OUT_COLS), lambda i: (0, 0)),
            pl.BlockSpec((OUT_COLS, EMB), lambda i: (0, 0)),
        ],
        out_specs=pl.BlockSpec((tile, EMB), lambda i: (i, 0)),
        out_shape=jax.ShapeDtypeStruct((t_pad, EMB), jnp.float32),
    )(g, sph, w, b_mat, p_mat)


def kernel(rbf, sph, idx_sph, weight):
    t = idx_sph.shape[0]
    tile = 1024
    per_w = -(-t // (_NW * _CHUNK)) * _CHUNK
    nchunks = per_w // _CHUNK
    t_pad = _NW * per_w
    if t_pad % tile:
        while (_NW * nchunks * _CHUNK) % tile:
            nchunks += 1
        t_pad = _NW * nchunks * _CHUNK

    idx_pad = jnp.zeros((t_pad,), jnp.int32).at[:t].set(idx_sph)
    sph_pad = jnp.zeros((t_pad, NUM_SPH), sph.dtype).at[:t].set(sph)

    g = _sc_gather(rbf, idx_pad, t_pad, nchunks)

    w = weight.reshape(NUM_RADIAL, OUT_COLS)
    b_mat = jnp.tile(jnp.eye(NUM_SPH, dtype=jnp.float32), (1, EMB))
    p_mat = jnp.repeat(jnp.eye(EMB, dtype=jnp.float32), NUM_SPH, axis=0)

    out = _tc_contract(g, sph_pad, w, b_mat, p_mat, t_pad, tile)
    return out[:t]


# speedup vs baseline: 28.4725x; 1.0321x over previous
import functools

import jax
import jax.numpy as jnp
from jax import lax
from jax.experimental import pallas as pl
from jax.experimental.pallas import tpu as pltpu
from jax.experimental.pallas import tpu_sc as plsc

NUM_RADIAL = 128
NUM_SPH = 8
EMB = 32
OUT_COLS = NUM_SPH * EMB

_NC = 2
_NS = 16
_NW = _NC * _NS
_CHUNK = 128


def _sc_gather(table, idx, t_pad, nchunks):
    mesh = plsc.VectorSubcoreMesh(core_axis_name="c", subcore_axis_name="s")

    @functools.partial(
        pl.kernel,
        mesh=mesh,
        out_type=jax.ShapeDtypeStruct((t_pad, NUM_RADIAL), jnp.float32),
        scratch_types=[
            pltpu.VMEM((_CHUNK,), jnp.int32),
            pltpu.VMEM((_CHUNK,), jnp.int32),
            pltpu.VMEM((_CHUNK, NUM_RADIAL), jnp.float32),
            pltpu.VMEM((_CHUNK, NUM_RADIAL), jnp.float32),
            pltpu.SemaphoreType.DMA,
            pltpu.SemaphoreType.DMA,
            pltpu.SemaphoreType.DMA,
            pltpu.SemaphoreType.DMA,
        ],
    )
    def k(table_hbm, idx_hbm, out_hbm, idx0, idx1, rows0, rows1,
          g0, g1, w0, w1):
        wid = lax.axis_index("s") * _NC + lax.axis_index("c")
        base = wid * nchunks

        def off(c):
            return (base + c) * _CHUNK

        def do_chunk(c, idxb, rowsb, gsem, wsem, drain_first):
            pltpu.sync_copy(idx_hbm.at[pl.ds(off(c), _CHUNK)], idxb)
            if drain_first:
                pltpu.make_async_copy(
                    rowsb, out_hbm.at[pl.ds(off(c), _CHUNK)], wsem).wait()
            pltpu.async_copy(table_hbm.at[idxb], rowsb, gsem).wait()
            pltpu.async_copy(rowsb, out_hbm.at[pl.ds(off(c), _CHUNK)], wsem)

        do_chunk(0, idx0, rows0, g0, w0, False)
        do_chunk(1, idx1, rows1, g1, w1, False)

        def body(j, carry):
            do_chunk(2 * j, idx0, rows0, g0, w0, True)
            do_chunk(2 * j + 1, idx1, rows1, g1, w1, True)
            return carry

        lax.fori_loop(1, nchunks // 2, body, 0, unroll=False)
        pltpu.make_async_copy(
            rows0, out_hbm.at[pl.ds(off(nchunks - 2), _CHUNK)], w0).wait()
        pltpu.make_async_copy(
            rows1, out_hbm.at[pl.ds(off(nchunks - 1), _CHUNK)], w1).wait()

    return k(table, idx)


def _tc_contract(g, sph, w, b_mat, p_mat, t_pad, tile):
    def body(g_ref, s_ref, w_ref, b_ref, p_ref, o_ref):
        h = jnp.dot(g_ref[...], w_ref[...], preferred_element_type=jnp.float32)
        srep = jnp.dot(s_ref[...], b_ref[...], preferred_element_type=jnp.float32)
        o_ref[...] = jnp.dot(h * srep, p_ref[...],
                             preferred_element_type=jnp.float32)

    return pl.pallas_call(
        body,
        grid=(t_pad // tile,),
        in_specs=[
            pl.BlockSpec((tile, NUM_RADIAL), lambda i: (i, 0)),
            pl.BlockSpec((tile, NUM_SPH), lambda i: (i, 0)),
            pl.BlockSpec((NUM_RADIAL, OUT_COLS), lambda i: (0, 0)),
            pl.BlockSpec((NUM_SPH, OUT_COLS), lambda i: (0, 0)),
            pl.BlockSpec((OUT_COLS, EMB), lambda i: (0, 0)),
        ],
        out_specs=pl.BlockSpec((tile, EMB), lambda i: (i, 0)),
        out_shape=jax.ShapeDtypeStruct((t_pad, EMB), jnp.float32),
    )(g, sph, w, b_mat, p_mat)


def kernel(rbf, sph, idx_sph, weight):
    t = idx_sph.shape[0]
    tile = 1024
    per_w = -(-t // (_NW * _CHUNK)) * _CHUNK
    nchunks = per_w // _CHUNK
    t_pad = _NW * per_w
    while nchunks % 2 or (_NW * nchunks * _CHUNK) % tile:
        nchunks += 1
    t_pad = _NW * nchunks * _CHUNK

    idx_pad = jnp.zeros((t_pad,), jnp.int32).at[:t].set(idx_sph)
    sph_pad = jnp.zeros((t_pad, NUM_SPH), sph.dtype).at[:t].set(sph)

    g = _sc_gather(rbf, idx_pad, t_pad, nchunks)

    w = weight.reshape(NUM_RADIAL, OUT_COLS)
    b_mat = jnp.tile(jnp.eye(NUM_SPH, dtype=jnp.float32), (1, EMB))
    p_mat = jnp.repeat(jnp.eye(EMB, dtype=jnp.float32), NUM_SPH, axis=0)

    out = _tc_contract(g, sph_pad, w, b_mat, p_mat, t_pad, tile)
    return out[:t]
